# baseline (speedup 1.0000x reference)
import functools

import jax
import jax.numpy as jnp
from jax import lax
from jax.experimental import pallas as pl
from jax.experimental.pallas import tpu as pltpu
from jax.experimental.pallas import tpu_sc as plsc


def _st_body(x_ref, we_ref, a0, a1, a2, a3, a4, wm_ref, b_ref,
             o0, o1, o2, o3, o4, xw_s):
    @pl.when(pl.program_id(0) == 0)
    def _():
        xw_s[...] = jnp.dot(x_ref[...], we_ref[...],
                            preferred_element_type=jnp.float32)

    for a_ref, o_ref in ((a0, o0), (a1, o1), (a2, o2), (a3, o3), (a4, o4)):
        h = jnp.dot(a_ref[...], xw_s[...],
                    preferred_element_type=jnp.float32)
        o_ref[...] = jnp.maximum(h, 0.0) @ wm_ref[...] + b_ref[...]


def _stage_tc(x, adj, w_enc, wm2, bias2, bm, nstreams=5):
    n, d = x.shape
    nh = n // nstreams
    nsteps = nh // bm
    out_sd = jax.ShapeDtypeStruct((nh, 2), jnp.float32)

    def mk_spec(k):
        return pl.BlockSpec((bm, n), lambda m, k=k: (m + k * nsteps, 0))

    return pl.pallas_call(
        _st_body,
        grid=(nsteps,),
        in_specs=[
            pl.BlockSpec((n, d), lambda m: (0, 0)),
            pl.BlockSpec((d, d), lambda m: (0, 0)),
        ] + [mk_spec(k) for k in range(nstreams)] + [
            pl.BlockSpec((d, 2), lambda m: (0, 0)),
            pl.BlockSpec((1, 2), lambda m: (0, 0)),
        ],
        out_specs=[
            pl.BlockSpec((bm, 2), lambda m: (m, 0)) for _ in range(nstreams)
        ],
        out_shape=[out_sd] * nstreams,
        scratch_shapes=[pltpu.VMEM((n, d), jnp.float32)],
        compiler_params=pltpu.CompilerParams(
            dimension_semantics=("arbitrary",),
        ),
    )(x, w_enc, *([adj] * nstreams), wm2, bias2)


def _make_sc_gather(n, e):
    info = plsc.get_sparse_core_info()
    nc, ns, nl = info.num_cores, info.num_subcores, info.num_lanes
    nw = nc * ns
    epw = e // nw
    mesh = plsc.VectorSubcoreMesh(core_axis_name="c", subcore_axis_name="s")

    @functools.partial(
        pl.kernel,
        out_type=jax.ShapeDtypeStruct((e,), jnp.float32),
        mesh=mesh,
        scratch_types=[
            pltpu.VMEM((2 * n,), jnp.float32),
            pltpu.VMEM((epw,), jnp.int32),
            pltpu.VMEM((epw,), jnp.int32),
            pltpu.VMEM((epw,), jnp.float32),
            pltpu.SemaphoreType.DMA,
        ],
        compiler_params=pltpu.CompilerParams(needs_layout_passes=False),
    )
    def sc_gather(st_hbm, eif_hbm, out_hbm, st_v, si_v, di_v, o_v, sem):
        wid = lax.axis_index("s") * nc + lax.axis_index("c")
        base = wid * epw
        c1 = pltpu.async_copy(st_hbm, st_v, sem)
        c2 = pltpu.async_copy(eif_hbm.at[pl.ds(base, epw)], si_v, sem)
        c3 = pltpu.async_copy(eif_hbm.at[pl.ds(e + base, epw)], di_v, sem)
        c1.wait()
        c2.wait()
        c3.wait()

        one = jnp.ones((nl,), jnp.int32)

        @plsc.parallel_loop(0, epw // nl, unroll=16)
        def body(i):
            off = i * nl
            sidx = si_v[pl.ds(off, nl)] * 2
            didx = di_v[pl.ds(off, nl)] * 2 + one
            sv = plsc.load_gather(st_v, [sidx])
            tv = plsc.load_gather(st_v, [didx])
            o_v[pl.ds(off, nl)] = sv + tv

        pltpu.sync_copy(o_v, out_hbm.at[pl.ds(base, epw)])

    return sc_gather


def kernel(x, adj, edge_index, W_enc, W_mlp, b_mlp):
    n, d = x.shape
    e = edge_index.shape[1]

    wm2 = jnp.concatenate([W_mlp[:d], W_mlp[d:]], axis=1)
    bias2 = jnp.concatenate([b_mlp, jnp.zeros_like(b_mlp)])
    bias2 = bias2.reshape(1, 2)

    st_parts = _stage_tc(x, adj, W_enc, wm2, bias2, bm=80)
    st_flat = jnp.concatenate(st_parts, axis=0).reshape(-1)
    ei_flat = edge_index.reshape(-1)

    out = _make_sc_gather(n, e)(st_flat, ei_flat)
    return out.reshape(e, 1)


# speedup vs baseline: 1.0290x; 1.0290x over previous
import functools

import jax
import jax.numpy as jnp
from jax import lax
from jax.experimental import pallas as pl
from jax.experimental.pallas import tpu as pltpu
from jax.experimental.pallas import tpu_sc as plsc


def _st_body(x_ref, we_ref, adj_t_ref, adj_b_ref, wm_ref, b_ref,
             ot_ref, ob_ref, xw_s):
    @pl.when(pl.program_id(0) == 0)
    def _():
        xw_s[...] = jnp.dot(x_ref[...], we_ref[...],
                            preferred_element_type=jnp.float32)

    ht = jnp.dot(adj_t_ref[...], xw_s[...],
                 preferred_element_type=jnp.float32)
    hb = jnp.dot(adj_b_ref[...], xw_s[...],
                 preferred_element_type=jnp.float32)
    ot_ref[...] = jnp.maximum(ht, 0.0) @ wm_ref[...] + b_ref[...]
    ob_ref[...] = jnp.maximum(hb, 0.0) @ wm_ref[...] + b_ref[...]


def _stage_tc(x, adj, w_enc, wm2, bias2, bm):
    n, d = x.shape
    nh = n // 2
    nsteps = nh // bm
    out_sd = jax.ShapeDtypeStruct((nh, 2), jnp.float32)
    return pl.pallas_call(
        _st_body,
        grid=(nsteps,),
        in_specs=[
            pl.BlockSpec((n, d), lambda m: (0, 0)),
            pl.BlockSpec((d, d), lambda m: (0, 0)),
            pl.BlockSpec((bm, n), lambda m: (m, 0)),
            pl.BlockSpec((bm, n), lambda m: (m + nsteps, 0)),
            pl.BlockSpec((d, 2), lambda m: (0, 0)),
            pl.BlockSpec((1, 2), lambda m: (0, 0)),
        ],
        out_specs=[
            pl.BlockSpec((bm, 2), lambda m: (m, 0)),
            pl.BlockSpec((bm, 2), lambda m: (m, 0)),
        ],
        out_shape=[out_sd, out_sd],
        scratch_shapes=[pltpu.VMEM((n, d), jnp.float32)],
        compiler_params=pltpu.CompilerParams(
            dimension_semantics=("arbitrary",),
        ),
    )(x, w_enc, adj, adj, wm2, bias2)


def _make_sc_gather(n, e):
    info = plsc.get_sparse_core_info()
    nc, ns, nl = info.num_cores, info.num_subcores, info.num_lanes
    nw = nc * ns
    epw = e // nw
    mesh = plsc.VectorSubcoreMesh(core_axis_name="c", subcore_axis_name="s")

    @functools.partial(
        pl.kernel,
        out_type=jax.ShapeDtypeStruct((e,), jnp.float32),
        mesh=mesh,
        scratch_types=[
            pltpu.VMEM((2 * n,), jnp.float32),
            pltpu.VMEM((epw,), jnp.int32),
            pltpu.VMEM((epw,), jnp.int32),
            pltpu.VMEM((epw,), jnp.float32),
            pltpu.SemaphoreType.DMA,
        ],
        compiler_params=pltpu.CompilerParams(needs_layout_passes=False),
    )
    def sc_gather(st_hbm, eif_hbm, out_hbm, st_v, si_v, di_v, o_v, sem):
        wid = lax.axis_index("s") * nc + lax.axis_index("c")
        base = wid * epw
        c1 = pltpu.async_copy(st_hbm, st_v, sem)
        c2 = pltpu.async_copy(eif_hbm.at[pl.ds(base, epw)], si_v, sem)
        c3 = pltpu.async_copy(eif_hbm.at[pl.ds(e + base, epw)], di_v, sem)
        c1.wait()
        c2.wait()
        c3.wait()

        one = jnp.ones((nl,), jnp.int32)

        @plsc.parallel_loop(0, epw // nl, unroll=16)
        def body(i):
            off = i * nl
            sidx = si_v[pl.ds(off, nl)] * 2
            didx = di_v[pl.ds(off, nl)] * 2 + one
            sv = plsc.load_gather(st_v, [sidx])
            tv = plsc.load_gather(st_v, [didx])
            o_v[pl.ds(off, nl)] = sv + tv

        pltpu.sync_copy(o_v, out_hbm.at[pl.ds(base, epw)])

    return sc_gather


def kernel(x, adj, edge_index, W_enc, W_mlp, b_mlp):
    n, d = x.shape
    e = edge_index.shape[1]

    wm2 = jnp.concatenate([W_mlp[:d], W_mlp[d:]], axis=1)
    bias2 = jnp.concatenate([b_mlp, jnp.zeros_like(b_mlp)])
    bias2 = bias2.reshape(1, 2)

    st_top, st_bot = _stage_tc(x, adj, W_enc, wm2, bias2, bm=200)
    st_flat = jnp.concatenate([st_top, st_bot], axis=0).reshape(-1)
    ei_flat = edge_index.reshape(-1)

    out = _make_sc_gather(n, e)(st_flat, ei_flat)
    return out.reshape(e, 1)
